# Initial kernel scaffold; baseline (speedup 1.0000x reference)
import jax
import jax.numpy as jnp
from jax.experimental import pallas as pl


def kernel(x, edge_index, params):
    raise NotImplementedError("write your pallas kernel here")


# speedup vs baseline: 6.3626x; 6.3626x over previous
import functools

import jax
import jax.numpy as jnp
from jax import lax
from jax.experimental import pallas as pl
from jax.experimental.pallas import tpu as pltpu
from jax.experimental.pallas import tpu_sc as plsc

N = 50000
IN = 6
H = 64
HH = 32
OUT = 4
L = 4
E = 800000

NC, NS = 2, 16
CH = 4
EDGES_ITER = CH * 128
AGG_ITERS = 98
E_PAD = NS * AGG_ITERS * EDGES_ITER
ROWS128 = E_PAD // 128
N_ACC = 50176
N_DUMMY = N
N_TILE = N_ACC // NS
W_OUT = 3128
W_OUT_LAST = N - (NS - 1) * W_OUT

CH_DEG = 4
DEG_ITERS = (ROWS128 // NC // NS) // CH_DEG

def _sc_aggregate_body(h_hbm, src_hbm, dst_hbm, zero_hbm, out_hbm,
                       isrc_v, idst_v, rows_v, acc_sh, zsem, gsem, ssem):
    c = lax.axis_index("c")
    s = lax.axis_index("s")

    pltpu.async_copy(zero_hbm, acc_sh.at[pl.ds(s * N_TILE, N_TILE)],
                     zsem).wait()
    plsc.subcore_barrier()

    def body(i, carry):
        rbase = s * (AGG_ITERS * CH) + i * CH
        pltpu.sync_copy(src_hbm.at[pl.ds(rbase, CH)], isrc_v)
        pltpu.sync_copy(dst_hbm.at[pl.ds(rbase, CH)], idst_v)
        gs = [pltpu.async_copy(h_hbm.at[c].at[isrc_v.at[j]],
                               rows_v.at[pl.ds(j * 128, 128)], gsem)
              for j in range(CH)]
        for d in gs:
            d.wait()
        ss = [pltpu.async_copy(rows_v.at[pl.ds(j * 128, 128)],
                               acc_sh.at[idst_v.at[j]], ssem, add=True)
              for j in range(CH)]
        for d in ss:
            d.wait()
        return carry

    lax.fori_loop(0, AGG_ITERS, body, 0)
    plsc.subcore_barrier()

    @pl.when(s < NS - 1)
    def _():
        pltpu.async_copy(acc_sh.at[pl.ds(s * W_OUT, W_OUT)],
                         out_hbm.at[c].at[pl.ds(s * W_OUT, W_OUT)],
                         zsem).wait()

    @pl.when(s == NS - 1)
    def _():
        pltpu.async_copy(acc_sh.at[pl.ds((NS - 1) * W_OUT, W_OUT_LAST)],
                         out_hbm.at[c].at[pl.ds((NS - 1) * W_OUT, W_OUT_LAST)],
                         zsem).wait()


def _sc_degree_body(dst_hbm, ones_hbm, zero_hbm, out_hbm,
                    idst_v, ones_v, acc_sh, zsem, ssem):
    c = lax.axis_index("c")
    s = lax.axis_index("s")

    pltpu.async_copy(zero_hbm, acc_sh.at[pl.ds(s * N_TILE, N_TILE)],
                     zsem).wait()
    pltpu.async_copy(ones_hbm, ones_v, ssem).wait()
    plsc.subcore_barrier()

    def body(i, carry):
        rbase = (c * NS + s) * (DEG_ITERS * CH_DEG) + i * CH_DEG
        pltpu.sync_copy(dst_hbm.at[pl.ds(rbase, CH_DEG)], idst_v)
        ss = [pltpu.async_copy(ones_v, acc_sh.at[idst_v.at[j]], ssem,
                               add=True)
              for j in range(CH_DEG)]
        for d in ss:
            d.wait()
        return carry

    lax.fori_loop(0, DEG_ITERS, body, 0)
    plsc.subcore_barrier()

    @pl.when(s < NS - 1)
    def _():
        pltpu.async_copy(acc_sh.at[pl.ds(s * W_OUT, W_OUT)],
                         out_hbm.at[c].at[pl.ds(s * W_OUT, W_OUT)],
                         zsem).wait()

    @pl.when(s == NS - 1)
    def _():
        pltpu.async_copy(acc_sh.at[pl.ds((NS - 1) * W_OUT, W_OUT_LAST)],
                         out_hbm.at[c].at[pl.ds((NS - 1) * W_OUT, W_OUT_LAST)],
                         zsem).wait()


@functools.cache
def _sc_kernels():
    mesh = plsc.VectorSubcoreMesh(core_axis_name="c", subcore_axis_name="s",
                                  num_cores=NC, num_subcores=NS)
    agg = pl.kernel(
        _sc_aggregate_body,
        out_type=jax.ShapeDtypeStruct((NC, N, HH), jnp.float32),
        mesh=mesh,
        scratch_types=[
            pltpu.VMEM((CH, 128), jnp.int32),
            pltpu.VMEM((CH, 128), jnp.int32),
            pltpu.VMEM((EDGES_ITER, HH), jnp.float32),
            pltpu.VMEM_SHARED((N_ACC, HH), jnp.float32),
            pltpu.SemaphoreType.DMA,
            pltpu.SemaphoreType.DMA,
            pltpu.SemaphoreType.DMA,
        ],
        compiler_params=pltpu.CompilerParams(use_tc_tiling_on_sc=False),
    )
    deg = pl.kernel(
        _sc_degree_body,
        out_type=jax.ShapeDtypeStruct((NC, N, 16), jnp.float32),
        mesh=mesh,
        scratch_types=[
            pltpu.VMEM((CH_DEG, 128), jnp.int32),
            pltpu.VMEM((128, 16), jnp.float32),
            pltpu.VMEM_SHARED((N_ACC, 16), jnp.float32),
            pltpu.SemaphoreType.DMA,
            pltpu.SemaphoreType.DMA,
        ],
        compiler_params=pltpu.CompilerParams(use_tc_tiling_on_sc=False),
    )
    return agg, deg


B = 2000


def _ln_relu(z, g, b):
    m = jnp.mean(z, axis=-1, keepdims=True)
    v = jnp.mean((z - m) * (z - m), axis=-1, keepdims=True)
    return jnp.maximum((z - m) / jnp.sqrt(v + 1e-5) * g + b, 0.0)


def _tc_input_body(x_ref, w_ref, b_ref, g_ref, bb_ref, out_ref):
    z = jnp.dot(x_ref[...], w_ref[...],
                preferred_element_type=jnp.float32) + b_ref[...]
    h = _ln_relu(z, g_ref[...], bb_ref[...])
    out_ref[0] = h[:, :HH]
    out_ref[1] = h[:, HH:]


def _tc_layer_body(agg_ref, h_ref, deg_ref, wl_ref, bl_ref, wr_ref,
                   g_ref, bb_ref, out_ref):
    hb = jnp.concatenate([h_ref[0], h_ref[1]], axis=-1)
    ab = jnp.concatenate([agg_ref[0], agg_ref[1]], axis=-1)
    deg = deg_ref[0][:, 0:1] + deg_ref[1][:, 0:1]
    inv = 1.0 / jnp.maximum(deg, 1.0)
    z = (jnp.dot(ab * inv, wl_ref[...], preferred_element_type=jnp.float32)
         + bl_ref[...]
         + jnp.dot(hb, wr_ref[...], preferred_element_type=jnp.float32))
    hn = hb + _ln_relu(z, g_ref[...], bb_ref[...])
    out_ref[0] = hn[:, :HH]
    out_ref[1] = hn[:, HH:]


def _tc_out_body(h_ref, w1_ref, b1_ref, g_ref, bb_ref, w2_ref, b2_ref,
                 w3_ref, b3_ref, out_ref):
    hb = jnp.concatenate([h_ref[0], h_ref[1]], axis=-1)
    z1 = _ln_relu(jnp.dot(hb, w1_ref[...],
                          preferred_element_type=jnp.float32) + b1_ref[...],
                  g_ref[...], bb_ref[...])
    z2 = jnp.maximum(jnp.dot(z1, w2_ref[...],
                             preferred_element_type=jnp.float32)
                     + b2_ref[...], 0.0)
    out_ref[...] = jnp.dot(z2, w3_ref[...],
                           preferred_element_type=jnp.float32) + b3_ref[...]


def _full(shape):
    return pl.BlockSpec(shape, lambda i: tuple(0 for _ in shape))


_GRID = N // B
_h2_spec = pl.BlockSpec((NC, B, HH), lambda i: (0, i, 0))


def _tc_input(x8, w8t, b, g, bb):
    return pl.pallas_call(
        _tc_input_body,
        grid=(_GRID,),
        in_specs=[pl.BlockSpec((B, 8), lambda i: (i, 0)),
                  _full((8, H)), _full((1, H)), _full((1, H)), _full((1, H))],
        out_specs=_h2_spec,
        out_shape=jax.ShapeDtypeStruct((NC, N, HH), jnp.float32),
    )(x8, w8t, b, g, bb)


def _tc_layer(agg2, h2, dega, wlt, bl, wrt, g, bb):
    return pl.pallas_call(
        _tc_layer_body,
        grid=(_GRID,),
        in_specs=[_h2_spec, _h2_spec,
                  pl.BlockSpec((NC, B, 16), lambda i: (0, i, 0)),
                  _full((H, H)), _full((1, H)), _full((H, H)),
                  _full((1, H)), _full((1, H))],
        out_specs=_h2_spec,
        out_shape=jax.ShapeDtypeStruct((NC, N, HH), jnp.float32),
    )(agg2, h2, dega, wlt, bl, wrt, g, bb)


def _tc_output(h2, w1t, b1, g, bb, w2t, b2, w3t, b3):
    return pl.pallas_call(
        _tc_out_body,
        grid=(_GRID,),
        in_specs=[_h2_spec,
                  _full((H, H)), _full((1, H)), _full((1, H)), _full((1, H)),
                  _full((H, HH)), _full((1, HH)),
                  _full((HH, OUT)), _full((1, OUT))],
        out_specs=pl.BlockSpec((B, OUT), lambda i: (i, 0)),
        out_shape=jax.ShapeDtypeStruct((N, OUT), jnp.float32),
    )(h2, w1t, b1, g, bb, w2t, b2, w3t, b3)


def kernel(x, edge_index, params):
    src = edge_index[0]
    dst = edge_index[1]
    pad = E_PAD - E
    src2 = jnp.concatenate([src, jnp.zeros((pad,), jnp.int32)]
                           ).reshape(ROWS128, 128)
    dst2 = jnp.concatenate([dst, jnp.full((pad,), N_DUMMY, jnp.int32)]
                           ).reshape(ROWS128, 128)

    x8 = jnp.pad(x, ((0, 0), (0, 8 - IN)))
    zero32 = jnp.zeros((N_TILE, HH), jnp.float32)
    zero16 = jnp.zeros((N_TILE, 16), jnp.float32)
    ones16 = jnp.ones((128, 16), jnp.float32)

    p = params
    row = lambda a: a.reshape(1, -1)

    sc_agg, sc_deg = _sc_kernels()
    h2 = _tc_input(x8, jnp.pad(p['in_W'], ((0, 0), (0, 8 - IN))).T,
                   row(p['in_b']), row(p['in_ln_g']), row(p['in_ln_b']))
    dega = sc_deg(dst2, ones16, zero16)
    for lp in p['layers']:
        agg2 = sc_agg(h2, src2, dst2, zero32)
        h2 = _tc_layer(agg2, h2, dega, lp['Wl'].T, row(lp['bl']),
                       lp['Wr'].T, row(lp['ln_g']), row(lp['ln_b']))
    return _tc_output(h2, p['out_W1'].T, row(p['out_b1']),
                      row(p['out_ln_g']), row(p['out_ln_b']),
                      p['out_W2'].T, row(p['out_b2']),
                      p['out_W3'].T, row(p['out_b3']))


# speedup vs baseline: 7.6133x; 1.1966x over previous
import functools

import jax
import jax.numpy as jnp
from jax import lax
from jax.experimental import pallas as pl
from jax.experimental.pallas import tpu as pltpu
from jax.experimental.pallas import tpu_sc as plsc

N = 50000
IN = 6
H = 64
HH = 32
OUT = 4
L = 4
E = 800000

NC, NS = 2, 16
CH = 4
NCHUNK = 49
AGG_OUTER = 8
EDGES_ITER = CH * 128
E_PAD = NS * AGG_OUTER * NCHUNK * 128
ROWS128 = E_PAD // 128
N_ACC = 50176
N_DUMMY = N
N_TILE = N_ACC // NS
W_OUT = 3128
W_OUT_LAST = N - (NS - 1) * W_OUT

CH_DEG = 4
DEG_ITERS = (ROWS128 // NC // NS) // CH_DEG

def _sc_aggregate_body(h_hbm, src_hbm, dst_hbm, zero_hbm, out_hbm,
                       isrc_v, idst_v, rows_v, acc_sh, zsem,
                       g0, g1, g2, g3, ssem):
    c = lax.axis_index("c")
    s = lax.axis_index("s")
    gsems = (g0, g1, g2, g3)

    pltpu.async_copy(zero_hbm, acc_sh.at[pl.ds(s * N_TILE, N_TILE)],
                     zsem).wait()
    plsc.subcore_barrier()

    def gather(chunk, slot):
        return pltpu.async_copy(h_hbm.at[c].at[isrc_v.at[chunk]],
                                rows_v.at[pl.ds(slot * 128, 128)],
                                gsems[slot])

    def scat(chunk, slot):
        return pltpu.async_copy(rows_v.at[pl.ds(slot * 128, 128)],
                                acc_sh.at[idst_v.at[chunk]], ssem, add=True)

    def outer(o, carry):
        rbase = s * (AGG_OUTER * NCHUNK) + o * NCHUNK
        pltpu.sync_copy(src_hbm.at[pl.ds(rbase, NCHUNK)], isrc_v)
        pltpu.sync_copy(dst_hbm.at[pl.ds(rbase, NCHUNK)], idst_v)

        def inner(i, carry2):
            k = i * CH
            gs = [gather(k + j, j) for j in range(CH)]
            ss = []
            for j in range(CH):
                gs[j].wait()
                ss.append(scat(k + j, j))
            for d in ss:
                d.wait()
            return carry2

        lax.fori_loop(0, NCHUNK // CH, inner, 0)
        gather(NCHUNK - 1, 0).wait()
        scat(NCHUNK - 1, 0).wait()
        return carry

    lax.fori_loop(0, AGG_OUTER, outer, 0)
    plsc.subcore_barrier()

    @pl.when(s < NS - 1)
    def _():
        pltpu.async_copy(acc_sh.at[pl.ds(s * W_OUT, W_OUT)],
                         out_hbm.at[c].at[pl.ds(s * W_OUT, W_OUT)],
                         zsem).wait()

    @pl.when(s == NS - 1)
    def _():
        pltpu.async_copy(acc_sh.at[pl.ds((NS - 1) * W_OUT, W_OUT_LAST)],
                         out_hbm.at[c].at[pl.ds((NS - 1) * W_OUT, W_OUT_LAST)],
                         zsem).wait()


def _sc_degree_body(dst_hbm, ones_hbm, zero_hbm, out_hbm,
                    idst_v, ones_v, acc_sh, zsem, ssem):
    c = lax.axis_index("c")
    s = lax.axis_index("s")

    pltpu.async_copy(zero_hbm, acc_sh.at[pl.ds(s * N_TILE, N_TILE)],
                     zsem).wait()
    pltpu.async_copy(ones_hbm, ones_v, ssem).wait()
    plsc.subcore_barrier()

    def body(i, carry):
        rbase = (c * NS + s) * (DEG_ITERS * CH_DEG) + i * CH_DEG
        pltpu.sync_copy(dst_hbm.at[pl.ds(rbase, CH_DEG)], idst_v)
        ss = [pltpu.async_copy(ones_v, acc_sh.at[idst_v.at[j]], ssem,
                               add=True)
              for j in range(CH_DEG)]
        for d in ss:
            d.wait()
        return carry

    lax.fori_loop(0, DEG_ITERS, body, 0)
    plsc.subcore_barrier()

    @pl.when(s < NS - 1)
    def _():
        pltpu.async_copy(acc_sh.at[pl.ds(s * W_OUT, W_OUT)],
                         out_hbm.at[c].at[pl.ds(s * W_OUT, W_OUT)],
                         zsem).wait()

    @pl.when(s == NS - 1)
    def _():
        pltpu.async_copy(acc_sh.at[pl.ds((NS - 1) * W_OUT, W_OUT_LAST)],
                         out_hbm.at[c].at[pl.ds((NS - 1) * W_OUT, W_OUT_LAST)],
                         zsem).wait()


@functools.cache
def _sc_kernels():
    mesh = plsc.VectorSubcoreMesh(core_axis_name="c", subcore_axis_name="s",
                                  num_cores=NC, num_subcores=NS)
    agg = pl.kernel(
        _sc_aggregate_body,
        out_type=jax.ShapeDtypeStruct((NC, N, HH), jnp.float32),
        mesh=mesh,
        scratch_types=[
            pltpu.VMEM((NCHUNK, 128), jnp.int32),
            pltpu.VMEM((NCHUNK, 128), jnp.int32),
            pltpu.VMEM((EDGES_ITER, HH), jnp.float32),
            pltpu.VMEM_SHARED((N_ACC, HH), jnp.float32),
            pltpu.SemaphoreType.DMA,
            pltpu.SemaphoreType.DMA,
            pltpu.SemaphoreType.DMA,
            pltpu.SemaphoreType.DMA,
            pltpu.SemaphoreType.DMA,
            pltpu.SemaphoreType.DMA,
        ],
        compiler_params=pltpu.CompilerParams(use_tc_tiling_on_sc=False),
    )
    deg = pl.kernel(
        _sc_degree_body,
        out_type=jax.ShapeDtypeStruct((NC, N, 16), jnp.float32),
        mesh=mesh,
        scratch_types=[
            pltpu.VMEM((CH_DEG, 128), jnp.int32),
            pltpu.VMEM((128, 16), jnp.float32),
            pltpu.VMEM_SHARED((N_ACC, 16), jnp.float32),
            pltpu.SemaphoreType.DMA,
            pltpu.SemaphoreType.DMA,
        ],
        compiler_params=pltpu.CompilerParams(use_tc_tiling_on_sc=False),
    )
    return agg, deg


B = 2000


def _ln_relu(z, g, b):
    m = jnp.mean(z, axis=-1, keepdims=True)
    v = jnp.mean((z - m) * (z - m), axis=-1, keepdims=True)
    return jnp.maximum((z - m) / jnp.sqrt(v + 1e-5) * g + b, 0.0)


def _tc_input_body(x_ref, w_ref, b_ref, g_ref, bb_ref, out_ref):
    z = jnp.dot(x_ref[...], w_ref[...],
                preferred_element_type=jnp.float32) + b_ref[...]
    h = _ln_relu(z, g_ref[...], bb_ref[...])
    out_ref[0] = h[:, :HH]
    out_ref[1] = h[:, HH:]


def _tc_layer_body(agg_ref, h_ref, deg_ref, wl_ref, bl_ref, wr_ref,
                   g_ref, bb_ref, out_ref):
    hb = jnp.concatenate([h_ref[0], h_ref[1]], axis=-1)
    ab = jnp.concatenate([agg_ref[0], agg_ref[1]], axis=-1)
    deg = deg_ref[0][:, 0:1] + deg_ref[1][:, 0:1]
    inv = 1.0 / jnp.maximum(deg, 1.0)
    z = (jnp.dot(ab * inv, wl_ref[...], preferred_element_type=jnp.float32)
         + bl_ref[...]
         + jnp.dot(hb, wr_ref[...], preferred_element_type=jnp.float32))
    hn = hb + _ln_relu(z, g_ref[...], bb_ref[...])
    out_ref[0] = hn[:, :HH]
    out_ref[1] = hn[:, HH:]


def _tc_out_body(h_ref, w1_ref, b1_ref, g_ref, bb_ref, w2_ref, b2_ref,
                 w3_ref, b3_ref, out_ref):
    hb = jnp.concatenate([h_ref[0], h_ref[1]], axis=-1)
    z1 = _ln_relu(jnp.dot(hb, w1_ref[...],
                          preferred_element_type=jnp.float32) + b1_ref[...],
                  g_ref[...], bb_ref[...])
    z2 = jnp.maximum(jnp.dot(z1, w2_ref[...],
                             preferred_element_type=jnp.float32)
                     + b2_ref[...], 0.0)
    out_ref[...] = jnp.dot(z2, w3_ref[...],
                           preferred_element_type=jnp.float32) + b3_ref[...]


def _full(shape):
    return pl.BlockSpec(shape, lambda i: tuple(0 for _ in shape))


_GRID = N // B
_h2_spec = pl.BlockSpec((NC, B, HH), lambda i: (0, i, 0))


def _tc_input(x8, w8t, b, g, bb):
    return pl.pallas_call(
        _tc_input_body,
        grid=(_GRID,),
        in_specs=[pl.BlockSpec((B, 8), lambda i: (i, 0)),
                  _full((8, H)), _full((1, H)), _full((1, H)), _full((1, H))],
        out_specs=_h2_spec,
        out_shape=jax.ShapeDtypeStruct((NC, N, HH), jnp.float32),
    )(x8, w8t, b, g, bb)


def _tc_layer(agg2, h2, dega, wlt, bl, wrt, g, bb):
    return pl.pallas_call(
        _tc_layer_body,
        grid=(_GRID,),
        in_specs=[_h2_spec, _h2_spec,
                  pl.BlockSpec((NC, B, 16), lambda i: (0, i, 0)),
                  _full((H, H)), _full((1, H)), _full((H, H)),
                  _full((1, H)), _full((1, H))],
        out_specs=_h2_spec,
        out_shape=jax.ShapeDtypeStruct((NC, N, HH), jnp.float32),
    )(agg2, h2, dega, wlt, bl, wrt, g, bb)


def _tc_output(h2, w1t, b1, g, bb, w2t, b2, w3t, b3):
    return pl.pallas_call(
        _tc_out_body,
        grid=(_GRID,),
        in_specs=[_h2_spec,
                  _full((H, H)), _full((1, H)), _full((1, H)), _full((1, H)),
                  _full((H, HH)), _full((1, HH)),
                  _full((HH, OUT)), _full((1, OUT))],
        out_specs=pl.BlockSpec((B, OUT), lambda i: (i, 0)),
        out_shape=jax.ShapeDtypeStruct((N, OUT), jnp.float32),
    )(h2, w1t, b1, g, bb, w2t, b2, w3t, b3)


def kernel(x, edge_index, params):
    src = edge_index[0]
    dst = edge_index[1]
    pad = E_PAD - E
    src2 = jnp.concatenate([src, jnp.zeros((pad,), jnp.int32)]
                           ).reshape(ROWS128, 128)
    dst2 = jnp.concatenate([dst, jnp.full((pad,), N_DUMMY, jnp.int32)]
                           ).reshape(ROWS128, 128)

    x8 = jnp.pad(x, ((0, 0), (0, 8 - IN)))
    zero32 = jnp.zeros((N_TILE, HH), jnp.float32)
    zero16 = jnp.zeros((N_TILE, 16), jnp.float32)
    ones16 = jnp.ones((128, 16), jnp.float32)

    p = params
    row = lambda a: a.reshape(1, -1)

    sc_agg, sc_deg = _sc_kernels()
    h2 = _tc_input(x8, jnp.pad(p['in_W'], ((0, 0), (0, 8 - IN))).T,
                   row(p['in_b']), row(p['in_ln_g']), row(p['in_ln_b']))
    dega = sc_deg(dst2, ones16, zero16)
    for lp in p['layers']:
        agg2 = sc_agg(h2, src2, dst2, zero32)
        h2 = _tc_layer(agg2, h2, dega, lp['Wl'].T, row(lp['bl']),
                       lp['Wr'].T, row(lp['ln_g']), row(lp['ln_b']))
    return _tc_output(h2, p['out_W1'].T, row(p['out_b1']),
                      row(p['out_ln_g']), row(p['out_ln_b']),
                      p['out_W2'].T, row(p['out_b2']),
                      p['out_W3'].T, row(p['out_b3']))


# speedup vs baseline: 7.8015x; 1.0247x over previous
import functools

import jax
import jax.numpy as jnp
from jax import lax
from jax.experimental import pallas as pl
from jax.experimental.pallas import tpu as pltpu
from jax.experimental.pallas import tpu_sc as plsc

N = 50000
NP = 51200
IN = 6
H = 64
HH = 32
OUT = 4
L = 4
E = 800000

NC, NS = 2, 16
SLOTS = 4
CPT = 392
BCH = 8
EDG_ROWS = 2 * CPT * NS
E_PAD = NS * CPT * 128
ROWS128 = E_PAD // 128
N_ACC = 50176
N_DUMMY = N
N_TILE = N_ACC // NS
W_OUT = 3128
W_OUT_LAST = N - (NS - 1) * W_OUT

CH_DEG = 4
DEG_ITERS = (ROWS128 // NC // NS) // CH_DEG

def _sc_aggregate_body(h_hbm, edg_hbm, zero_hbm, out_hbm,
                       idx_v, rows_v, acc_sh,
                       zsem, g0, g1, g2, g3, s0, s1, s2, s3):
    c = lax.axis_index("c")
    s = lax.axis_index("s")
    gsems = (g0, g1, g2, g3)
    ssems = (s0, s1, s2, s3)

    pltpu.async_copy(zero_hbm, acc_sh.at[pl.ds(s * N_TILE, N_TILE)],
                     zsem).wait()
    plsc.subcore_barrier()

    base = s * (2 * CPT)

    def gather(q, slot):
        return pltpu.async_copy(h_hbm.at[c].at[idx_v.at[2 * q]],
                                rows_v.at[pl.ds(slot * 128, 128)],
                                gsems[slot])

    def scat(q, slot):
        return pltpu.async_copy(rows_v.at[pl.ds(slot * 128, 128)],
                                acc_sh.at[idx_v.at[2 * q + 1]],
                                ssems[slot], add=True)

    def outer(b, carry):
        pltpu.sync_copy(edg_hbm.at[pl.ds(base + 2 * BCH * b, 2 * BCH)],
                        idx_v)
        gd = [None] * BCH
        sd = [None] * BCH
        for q in range(BCH):
            if q >= SLOTS:
                sd[q - SLOTS].wait()
            gd[q] = gather(q, q % SLOTS)
            if q >= 3:
                gd[q - 3].wait()
                sd[q - 3] = scat(q - 3, (q - 3) % SLOTS)
        for qq in range(BCH - 3, BCH):
            gd[qq].wait()
            sd[qq] = scat(qq, qq % SLOTS)
        for qq in range(BCH - SLOTS, BCH):
            sd[qq].wait()
        return carry

    lax.fori_loop(0, CPT // BCH, outer, 0)
    plsc.subcore_barrier()

    @pl.when(s < NS - 1)
    def _():
        pltpu.async_copy(acc_sh.at[pl.ds(s * W_OUT, W_OUT)],
                         out_hbm.at[c].at[pl.ds(s * W_OUT, W_OUT)],
                         zsem).wait()

    @pl.when(s == NS - 1)
    def _():
        pltpu.async_copy(acc_sh.at[pl.ds((NS - 1) * W_OUT, W_OUT_LAST)],
                         out_hbm.at[c].at[pl.ds((NS - 1) * W_OUT, W_OUT_LAST)],
                         zsem).wait()


def _sc_degree_body(dst_hbm, ones_hbm, zero_hbm, out_hbm,
                    idst_v, ones_v, acc_sh, zsem, ssem):
    c = lax.axis_index("c")
    s = lax.axis_index("s")

    pltpu.async_copy(zero_hbm, acc_sh.at[pl.ds(s * N_TILE, N_TILE)],
                     zsem).wait()
    pltpu.async_copy(ones_hbm, ones_v, ssem).wait()
    plsc.subcore_barrier()

    def body(i, carry):
        rbase = (c * NS + s) * (DEG_ITERS * CH_DEG) + i * CH_DEG
        pltpu.sync_copy(dst_hbm.at[pl.ds(rbase, CH_DEG)], idst_v)
        ss = [pltpu.async_copy(ones_v, acc_sh.at[idst_v.at[j]], ssem,
                               add=True)
              for j in range(CH_DEG)]
        for d in ss:
            d.wait()
        return carry

    lax.fori_loop(0, DEG_ITERS, body, 0)
    plsc.subcore_barrier()

    @pl.when(s < NS - 1)
    def _():
        pltpu.async_copy(acc_sh.at[pl.ds(s * W_OUT, W_OUT)],
                         out_hbm.at[c].at[pl.ds(s * W_OUT, W_OUT)],
                         zsem).wait()

    @pl.when(s == NS - 1)
    def _():
        pltpu.async_copy(acc_sh.at[pl.ds((NS - 1) * W_OUT, W_OUT_LAST)],
                         out_hbm.at[c].at[pl.ds((NS - 1) * W_OUT, W_OUT_LAST)],
                         zsem).wait()


@functools.cache
def _sc_kernels():
    mesh = plsc.VectorSubcoreMesh(core_axis_name="c", subcore_axis_name="s",
                                  num_cores=NC, num_subcores=NS)
    agg = pl.kernel(
        _sc_aggregate_body,
        out_type=jax.ShapeDtypeStruct((NC, N, HH), jnp.float32),
        mesh=mesh,
        scratch_types=[
            pltpu.VMEM((2 * BCH, 128), jnp.int32),
            pltpu.VMEM((SLOTS * 128, HH), jnp.float32),
            pltpu.VMEM_SHARED((N_ACC, HH), jnp.float32),
        ] + [pltpu.SemaphoreType.DMA] * 9,
        compiler_params=pltpu.CompilerParams(use_tc_tiling_on_sc=False),
    )
    deg = pl.kernel(
        _sc_degree_body,
        out_type=jax.ShapeDtypeStruct((NC, N, HH), jnp.float32),
        mesh=mesh,
        scratch_types=[
            pltpu.VMEM((CH_DEG, 128), jnp.int32),
            pltpu.VMEM((128, HH), jnp.float32),
            pltpu.VMEM_SHARED((N_ACC, HH), jnp.float32),
            pltpu.SemaphoreType.DMA,
            pltpu.SemaphoreType.DMA,
        ],
        compiler_params=pltpu.CompilerParams(use_tc_tiling_on_sc=False),
    )
    return agg, deg


B = 2000


def _ln_relu(z, g, b):
    m = jnp.mean(z, axis=-1, keepdims=True)
    v = jnp.mean((z - m) * (z - m), axis=-1, keepdims=True)
    return jnp.maximum((z - m) / jnp.sqrt(v + 1e-5) * g + b, 0.0)


def _tc_input_body(x_ref, w_ref, b_ref, g_ref, bb_ref, out_ref):
    z = jnp.dot(x_ref[...], w_ref[...],
                preferred_element_type=jnp.float32) + b_ref[...]
    h = _ln_relu(z, g_ref[...], bb_ref[...])
    out_ref[0] = h[:, :HH]
    out_ref[1] = h[:, HH:]


def _tc_layer_body(agg_ref, h_ref, deg_ref, wl_ref, bl_ref, wr_ref,
                   g_ref, bb_ref, out_ref):
    hb = jnp.concatenate([h_ref[0], h_ref[1]], axis=-1)
    ab = jnp.concatenate([agg_ref[0], agg_ref[1]], axis=-1)
    deg = deg_ref[0][:, 0:1] + deg_ref[1][:, 0:1]
    inv = 1.0 / jnp.maximum(deg, 1.0)
    z = (jnp.dot(ab * inv, wl_ref[...], preferred_element_type=jnp.float32)
         + bl_ref[...]
         + jnp.dot(hb, wr_ref[...], preferred_element_type=jnp.float32))
    hn = hb + _ln_relu(z, g_ref[...], bb_ref[...])
    out_ref[0] = hn[:, :HH]
    out_ref[1] = hn[:, HH:]


def _tc_out_body(h_ref, w1_ref, b1_ref, g_ref, bb_ref, w2_ref, b2_ref,
                 w3_ref, b3_ref, out_ref):
    hb = jnp.concatenate([h_ref[0], h_ref[1]], axis=-1)
    z1 = _ln_relu(jnp.dot(hb, w1_ref[...],
                          preferred_element_type=jnp.float32) + b1_ref[...],
                  g_ref[...], bb_ref[...])
    z2 = jnp.maximum(jnp.dot(z1, w2_ref[...],
                             preferred_element_type=jnp.float32)
                     + b2_ref[...], 0.0)
    out_ref[...] = jnp.dot(z2, w3_ref[...],
                           preferred_element_type=jnp.float32) + b3_ref[...]


def _full(shape):
    return pl.BlockSpec(shape, lambda i: tuple(0 for _ in shape))


_GRID = N // B
_h2_spec = pl.BlockSpec((NC, B, HH), lambda i: (0, i, 0))


def _tc_input(x8, w8t, b, g, bb):
    return pl.pallas_call(
        _tc_input_body,
        grid=(_GRID,),
        in_specs=[pl.BlockSpec((B, 8), lambda i: (i, 0)),
                  _full((8, H)), _full((1, H)), _full((1, H)), _full((1, H))],
        out_specs=_h2_spec,
        out_shape=jax.ShapeDtypeStruct((NC, N, HH), jnp.float32),
    )(x8, w8t, b, g, bb)


def _tc_layer(agg2, h2, dega, wlt, bl, wrt, g, bb):
    return pl.pallas_call(
        _tc_layer_body,
        grid=(_GRID,),
        in_specs=[_h2_spec, _h2_spec, _h2_spec,
                  _full((H, H)), _full((1, H)), _full((H, H)),
                  _full((1, H)), _full((1, H))],
        out_specs=_h2_spec,
        out_shape=jax.ShapeDtypeStruct((NC, N, HH), jnp.float32),
    )(agg2, h2, dega, wlt, bl, wrt, g, bb)


def _tc_output(h2, w1t, b1, g, bb, w2t, b2, w3t, b3):
    return pl.pallas_call(
        _tc_out_body,
        grid=(_GRID,),
        in_specs=[_h2_spec,
                  _full((H, H)), _full((1, H)), _full((1, H)), _full((1, H)),
                  _full((H, HH)), _full((1, HH)),
                  _full((HH, OUT)), _full((1, OUT))],
        out_specs=pl.BlockSpec((B, OUT), lambda i: (i, 0)),
        out_shape=jax.ShapeDtypeStruct((N, OUT), jnp.float32),
    )(h2, w1t, b1, g, bb, w2t, b2, w3t, b3)


def kernel(x, edge_index, params):
    src = edge_index[0]
    dst = edge_index[1]
    pad = E_PAD - E
    src2 = jnp.concatenate([src, jnp.zeros((pad,), jnp.int32)]
                           ).reshape(ROWS128, 128)
    dst2 = jnp.concatenate([dst, jnp.full((pad,), N_DUMMY, jnp.int32)]
                           ).reshape(ROWS128, 128)
    edg2 = jnp.stack([src2, dst2], axis=1).reshape(EDG_ROWS, 128)

    x8 = jnp.pad(x, ((0, 0), (0, 8 - IN)))
    zero32 = jnp.zeros((N_TILE, HH), jnp.float32)
    ones32 = jnp.ones((128, HH), jnp.float32)

    p = params
    row = lambda a: a.reshape(1, -1)

    sc_agg, sc_deg = _sc_kernels()
    h2 = _tc_input(x8, jnp.pad(p['in_W'], ((0, 0), (0, 8 - IN))).T,
                   row(p['in_b']), row(p['in_ln_g']), row(p['in_ln_b']))
    dega = sc_deg(dst2, ones32, zero32)
    for lp in p['layers']:
        agg2 = sc_agg(h2, edg2, zero32)
        h2 = _tc_layer(agg2, h2, dega, lp['Wl'].T, row(lp['bl']),
                       lp['Wr'].T, row(lp['ln_g']), row(lp['ln_b']))
    return _tc_output(h2, p['out_W1'].T, row(p['out_b1']),
                      row(p['out_ln_g']), row(p['out_ln_b']),
                      p['out_W2'].T, row(p['out_b2']),
                      p['out_W3'].T, row(p['out_b3']))


# speedup vs baseline: 7.9100x; 1.0139x over previous
import functools

import jax
import jax.numpy as jnp
from jax import lax
from jax.experimental import pallas as pl
from jax.experimental.pallas import tpu as pltpu
from jax.experimental.pallas import tpu_sc as plsc

N = 50000
NP = 51200
IN = 6
H = 64
HH = 32
OUT = 4
L = 4
E = 800000

NC, NS = 2, 16
SLOTS = 4
CPT = 392
BCH = 8
EDG_ROWS = 2 * CPT * NS
E_PAD = NS * CPT * 128
ROWS128 = E_PAD // 128
N_ACC = 50176
N_DUMMY = N
N_TILE = N_ACC // NS
W_OUT = 3128
W_OUT_LAST = N - (NS - 1) * W_OUT

CH_DEG = 4
DEG_ITERS = (ROWS128 // NC // NS) // CH_DEG

def _sc_aggregate_body(h_hbm, edg_hbm, zero_hbm, out_hbm,
                       idx_v, rows_v, acc_sh,
                       zsem, g0, g1, g2, g3, s0, s1, s2, s3):
    c = lax.axis_index("c")
    s = lax.axis_index("s")
    gsems = (g0, g1, g2, g3)
    ssems = (s0, s1, s2, s3)

    pltpu.async_copy(zero_hbm, acc_sh.at[pl.ds(s * N_TILE, N_TILE)],
                     zsem).wait()
    plsc.subcore_barrier()

    base = s * (2 * CPT)

    def gather(q, slot):
        return pltpu.async_copy(h_hbm.at[c].at[idx_v.at[2 * q]],
                                rows_v.at[pl.ds(slot * 128, 128)],
                                gsems[slot])

    def scat(q, slot):
        return pltpu.async_copy(rows_v.at[pl.ds(slot * 128, 128)],
                                acc_sh.at[idx_v.at[2 * q + 1]],
                                ssems[slot], add=True)

    def outer(b, carry):
        pltpu.sync_copy(edg_hbm.at[pl.ds(base + 2 * BCH * b, 2 * BCH)],
                        idx_v)
        gd = [None] * BCH
        sd = [None] * BCH
        for q in range(BCH):
            if q >= SLOTS:
                sd[q - SLOTS].wait()
            gd[q] = gather(q, q % SLOTS)
            if q >= 3:
                gd[q - 3].wait()
                sd[q - 3] = scat(q - 3, (q - 3) % SLOTS)
        for qq in range(BCH - 3, BCH):
            gd[qq].wait()
            sd[qq] = scat(qq, qq % SLOTS)
        for qq in range(BCH - SLOTS, BCH):
            sd[qq].wait()
        return carry

    lax.fori_loop(0, CPT // BCH, outer, 0)
    plsc.subcore_barrier()

    @pl.when(s < NS - 1)
    def _():
        pltpu.async_copy(acc_sh.at[pl.ds(s * W_OUT, W_OUT)],
                         out_hbm.at[c].at[pl.ds(s * W_OUT, W_OUT)],
                         zsem).wait()

    @pl.when(s == NS - 1)
    def _():
        pltpu.async_copy(acc_sh.at[pl.ds((NS - 1) * W_OUT, W_OUT_LAST)],
                         out_hbm.at[c].at[pl.ds((NS - 1) * W_OUT, W_OUT_LAST)],
                         zsem).wait()


def _sc_degree_body(dst_hbm, ones_hbm, zero_hbm, out_hbm,
                    idst_v, ones_v, acc_sh, zsem, ssem):
    c = lax.axis_index("c")
    s = lax.axis_index("s")

    pltpu.async_copy(zero_hbm, acc_sh.at[pl.ds(s * N_TILE, N_TILE)],
                     zsem).wait()
    pltpu.async_copy(ones_hbm, ones_v, ssem).wait()
    plsc.subcore_barrier()

    def body(i, carry):
        rbase = (c * NS + s) * (DEG_ITERS * CH_DEG) + i * CH_DEG
        pltpu.sync_copy(dst_hbm.at[pl.ds(rbase, CH_DEG)], idst_v)
        ss = [pltpu.async_copy(ones_v, acc_sh.at[idst_v.at[j]], ssem,
                               add=True)
              for j in range(CH_DEG)]
        for d in ss:
            d.wait()
        return carry

    lax.fori_loop(0, DEG_ITERS, body, 0)
    plsc.subcore_barrier()

    @pl.when(s < NS - 1)
    def _():
        pltpu.async_copy(acc_sh.at[pl.ds(s * W_OUT, W_OUT)],
                         out_hbm.at[c].at[pl.ds(s * W_OUT, W_OUT)],
                         zsem).wait()

    @pl.when(s == NS - 1)
    def _():
        pltpu.async_copy(acc_sh.at[pl.ds((NS - 1) * W_OUT, W_OUT_LAST)],
                         out_hbm.at[c].at[pl.ds((NS - 1) * W_OUT, W_OUT_LAST)],
                         zsem).wait()


@functools.cache
def _sc_kernels():
    mesh = plsc.VectorSubcoreMesh(core_axis_name="c", subcore_axis_name="s",
                                  num_cores=NC, num_subcores=NS)
    agg = pl.kernel(
        _sc_aggregate_body,
        out_type=jax.ShapeDtypeStruct((NC, N, HH), jnp.float32),
        mesh=mesh,
        scratch_types=[
            pltpu.VMEM((2 * BCH, 128), jnp.int32),
            pltpu.VMEM((SLOTS * 128, HH), jnp.float32),
            pltpu.VMEM_SHARED((N_ACC, HH), jnp.float32),
        ] + [pltpu.SemaphoreType.DMA] * 9,
        compiler_params=pltpu.CompilerParams(use_tc_tiling_on_sc=False),
    )
    deg = pl.kernel(
        _sc_degree_body,
        out_type=jax.ShapeDtypeStruct((NC, N, 16), jnp.float32),
        mesh=mesh,
        scratch_types=[
            pltpu.VMEM((CH_DEG, 128), jnp.int32),
            pltpu.VMEM((128, 16), jnp.float32),
            pltpu.VMEM_SHARED((N_ACC, 16), jnp.float32),
            pltpu.SemaphoreType.DMA,
            pltpu.SemaphoreType.DMA,
        ],
        compiler_params=pltpu.CompilerParams(use_tc_tiling_on_sc=False),
    )
    return agg, deg


B = 2000


def _ln_relu(z, g, b):
    m = jnp.mean(z, axis=-1, keepdims=True)
    v = jnp.mean((z - m) * (z - m), axis=-1, keepdims=True)
    return jnp.maximum((z - m) / jnp.sqrt(v + 1e-5) * g + b, 0.0)


def _tc_input_body(x_ref, w_ref, b_ref, g_ref, bb_ref, out_ref):
    z = jnp.dot(x_ref[...], w_ref[...],
                preferred_element_type=jnp.float32) + b_ref[...]
    h = _ln_relu(z, g_ref[...], bb_ref[...])
    out_ref[0] = h[:, :HH]
    out_ref[1] = h[:, HH:]


def _tc_layer_body(agg_ref, h_ref, deg_ref, wl_ref, bl_ref, wr_ref,
                   g_ref, bb_ref, out_ref):
    hb = jnp.concatenate([h_ref[0], h_ref[1]], axis=-1)
    ab = jnp.concatenate([agg_ref[0], agg_ref[1]], axis=-1)
    deg = deg_ref[0][:, 0:1] + deg_ref[1][:, 0:1]
    inv = 1.0 / jnp.maximum(deg, 1.0)
    z = (jnp.dot(ab * inv, wl_ref[...], preferred_element_type=jnp.float32)
         + bl_ref[...]
         + jnp.dot(hb, wr_ref[...], preferred_element_type=jnp.float32))
    hn = hb + _ln_relu(z, g_ref[...], bb_ref[...])
    out_ref[0] = hn[:, :HH]
    out_ref[1] = hn[:, HH:]


def _tc_out_body(h_ref, w1_ref, b1_ref, g_ref, bb_ref, w2_ref, b2_ref,
                 w3_ref, b3_ref, out_ref):
    hb = jnp.concatenate([h_ref[0], h_ref[1]], axis=-1)
    z1 = _ln_relu(jnp.dot(hb, w1_ref[...],
                          preferred_element_type=jnp.float32) + b1_ref[...],
                  g_ref[...], bb_ref[...])
    z2 = jnp.maximum(jnp.dot(z1, w2_ref[...],
                             preferred_element_type=jnp.float32)
                     + b2_ref[...], 0.0)
    out_ref[...] = jnp.dot(z2, w3_ref[...],
                           preferred_element_type=jnp.float32) + b3_ref[...]


def _full(shape):
    return pl.BlockSpec(shape, lambda i: tuple(0 for _ in shape))


_GRID = N // B
_h2_spec = pl.BlockSpec((NC, B, HH), lambda i: (0, i, 0))


def _tc_input(x8, w8t, b, g, bb):
    return pl.pallas_call(
        _tc_input_body,
        grid=(_GRID,),
        in_specs=[pl.BlockSpec((B, 8), lambda i: (i, 0)),
                  _full((8, H)), _full((1, H)), _full((1, H)), _full((1, H))],
        out_specs=_h2_spec,
        out_shape=jax.ShapeDtypeStruct((NC, N, HH), jnp.float32),
    )(x8, w8t, b, g, bb)


def _tc_layer(agg2, h2, dega, wlt, bl, wrt, g, bb):
    return pl.pallas_call(
        _tc_layer_body,
        grid=(_GRID,),
        in_specs=[_h2_spec, _h2_spec,
                  pl.BlockSpec((NC, B, 16), lambda i: (0, i, 0)),
                  _full((H, H)), _full((1, H)), _full((H, H)),
                  _full((1, H)), _full((1, H))],
        out_specs=_h2_spec,
        out_shape=jax.ShapeDtypeStruct((NC, N, HH), jnp.float32),
    )(agg2, h2, dega, wlt, bl, wrt, g, bb)


def _tc_output(h2, w1t, b1, g, bb, w2t, b2, w3t, b3):
    return pl.pallas_call(
        _tc_out_body,
        grid=(_GRID,),
        in_specs=[_h2_spec,
                  _full((H, H)), _full((1, H)), _full((1, H)), _full((1, H)),
                  _full((H, HH)), _full((1, HH)),
                  _full((HH, OUT)), _full((1, OUT))],
        out_specs=pl.BlockSpec((B, OUT), lambda i: (i, 0)),
        out_shape=jax.ShapeDtypeStruct((N, OUT), jnp.float32),
    )(h2, w1t, b1, g, bb, w2t, b2, w3t, b3)


def kernel(x, edge_index, params):
    src = edge_index[0]
    dst = edge_index[1]
    pad = E_PAD - E
    src2 = jnp.concatenate([src, jnp.zeros((pad,), jnp.int32)]
                           ).reshape(ROWS128, 128)
    dst2 = jnp.concatenate([dst, jnp.full((pad,), N_DUMMY, jnp.int32)]
                           ).reshape(ROWS128, 128)
    edg2 = jnp.stack([src2, dst2], axis=1).reshape(EDG_ROWS, 128)

    x8 = jnp.pad(x, ((0, 0), (0, 8 - IN)))
    zero32 = jnp.zeros((N_TILE, HH), jnp.float32)
    zero16 = jnp.zeros((N_TILE, 16), jnp.float32)
    ones16 = jnp.ones((128, 16), jnp.float32)

    p = params
    row = lambda a: a.reshape(1, -1)

    sc_agg, sc_deg = _sc_kernels()
    h2 = _tc_input(x8, jnp.pad(p['in_W'], ((0, 0), (0, 8 - IN))).T,
                   row(p['in_b']), row(p['in_ln_g']), row(p['in_ln_b']))
    dega = sc_deg(dst2, ones16, zero16)
    for lp in p['layers']:
        agg2 = sc_agg(h2, edg2, zero32)
        h2 = _tc_layer(agg2, h2, dega, lp['Wl'].T, row(lp['bl']),
                       lp['Wr'].T, row(lp['ln_g']), row(lp['ln_b']))
    return _tc_output(h2, p['out_W1'].T, row(p['out_b1']),
                      row(p['out_ln_g']), row(p['out_ln_b']),
                      p['out_W2'].T, row(p['out_b2']),
                      p['out_W3'].T, row(p['out_b3']))


# speedup vs baseline: 7.9121x; 1.0003x over previous
import functools

import jax
import jax.numpy as jnp
from jax import lax
from jax.experimental import pallas as pl
from jax.experimental.pallas import tpu as pltpu
from jax.experimental.pallas import tpu_sc as plsc

N = 50000
IN = 6
H = 64
HH = 32
OUT = 4
L = 4
E = 800000

NC, NS = 2, 16
SLOTS = 4
CPT = 392
BCH = 8
EDG_ROWS = 2 * CPT * NS
E_PAD = NS * CPT * 128
ROWS128 = E_PAD // 128
N_ACC = 50176
N_DUMMY = N
N_TILE = N_ACC // NS
W_OUT = 3128
W_OUT_LAST = N - (NS - 1) * W_OUT

CH_DEG = 4
DEG_ITERS = (ROWS128 // NC // NS) // CH_DEG

def _sc_aggregate_body(h_hbm, edg_hbm, zero_hbm, out_hbm,
                       idx_v, rows_v, acc_sh,
                       zsem, g0, g1, g2, g3, s0, s1, s2, s3):
    c = lax.axis_index("c")
    s = lax.axis_index("s")
    gsems = (g0, g1, g2, g3)
    ssems = (s0, s1, s2, s3)

    pltpu.async_copy(zero_hbm, acc_sh.at[pl.ds(s * N_TILE, N_TILE)],
                     zsem).wait()
    plsc.subcore_barrier()

    base = s * (2 * CPT)

    def gather(q, slot):
        return pltpu.async_copy(h_hbm.at[c].at[idx_v.at[2 * q]],
                                rows_v.at[pl.ds(slot * 128, 128)],
                                gsems[slot])

    def scat(q, slot):
        return pltpu.async_copy(rows_v.at[pl.ds(slot * 128, 128)],
                                acc_sh.at[idx_v.at[2 * q + 1]],
                                ssems[slot], add=True)

    def outer(b, carry):
        pltpu.sync_copy(edg_hbm.at[pl.ds(base + 2 * BCH * b, 2 * BCH)],
                        idx_v)
        gd = [None] * BCH
        sd = [None] * BCH
        for q in range(BCH):
            if q >= SLOTS:
                sd[q - SLOTS].wait()
            gd[q] = gather(q, q % SLOTS)
            if q >= 3:
                gd[q - 3].wait()
                sd[q - 3] = scat(q - 3, (q - 3) % SLOTS)
        for qq in range(BCH - 3, BCH):
            gd[qq].wait()
            sd[qq] = scat(qq, qq % SLOTS)
        for qq in range(BCH - SLOTS, BCH):
            sd[qq].wait()
        return carry

    lax.fori_loop(0, CPT // BCH, outer, 0)
    plsc.subcore_barrier()

    @pl.when(s < NS - 1)
    def _():
        pltpu.async_copy(acc_sh.at[pl.ds(s * W_OUT, W_OUT)],
                         out_hbm.at[c].at[pl.ds(s * W_OUT, W_OUT)],
                         zsem).wait()

    @pl.when(s == NS - 1)
    def _():
        pltpu.async_copy(acc_sh.at[pl.ds((NS - 1) * W_OUT, W_OUT_LAST)],
                         out_hbm.at[c].at[pl.ds((NS - 1) * W_OUT, W_OUT_LAST)],
                         zsem).wait()


def _sc_degree_body(dst_hbm, ones_hbm, zero_hbm, out_hbm,
                    idst_v, ones_v, acc_sh, zsem, ssem):
    c = lax.axis_index("c")
    s = lax.axis_index("s")

    pltpu.async_copy(zero_hbm, acc_sh.at[pl.ds(s * N_TILE, N_TILE)],
                     zsem).wait()
    pltpu.async_copy(ones_hbm, ones_v, ssem).wait()
    plsc.subcore_barrier()

    def body(i, carry):
        rbase = (c * NS + s) * (DEG_ITERS * CH_DEG) + i * CH_DEG
        pltpu.sync_copy(dst_hbm.at[pl.ds(rbase, CH_DEG)], idst_v)
        ss = [pltpu.async_copy(ones_v, acc_sh.at[idst_v.at[j]], ssem,
                               add=True)
              for j in range(CH_DEG)]
        for d in ss:
            d.wait()
        return carry

    lax.fori_loop(0, DEG_ITERS, body, 0)
    plsc.subcore_barrier()

    @pl.when(s < NS - 1)
    def _():
        pltpu.async_copy(acc_sh.at[pl.ds(s * W_OUT, W_OUT)],
                         out_hbm.at[c].at[pl.ds(s * W_OUT, W_OUT)],
                         zsem).wait()

    @pl.when(s == NS - 1)
    def _():
        pltpu.async_copy(acc_sh.at[pl.ds((NS - 1) * W_OUT, W_OUT_LAST)],
                         out_hbm.at[c].at[pl.ds((NS - 1) * W_OUT, W_OUT_LAST)],
                         zsem).wait()


@functools.cache
def _sc_kernels():
    mesh = plsc.VectorSubcoreMesh(core_axis_name="c", subcore_axis_name="s",
                                  num_cores=NC, num_subcores=NS)
    agg = pl.kernel(
        _sc_aggregate_body,
        out_type=jax.ShapeDtypeStruct((NC, N, HH), jnp.float32),
        mesh=mesh,
        scratch_types=[
            pltpu.VMEM((2 * BCH, 128), jnp.int32),
            pltpu.VMEM((SLOTS * 128, HH), jnp.float32),
            pltpu.VMEM_SHARED((N_ACC, HH), jnp.float32),
        ] + [pltpu.SemaphoreType.DMA] * 9,
        compiler_params=pltpu.CompilerParams(use_tc_tiling_on_sc=False),
    )
    deg = pl.kernel(
        _sc_degree_body,
        out_type=jax.ShapeDtypeStruct((NC, N, 16), jnp.float32),
        mesh=mesh,
        scratch_types=[
            pltpu.VMEM((CH_DEG, 128), jnp.int32),
            pltpu.VMEM((128, 16), jnp.float32),
            pltpu.VMEM_SHARED((N_ACC, 16), jnp.float32),
            pltpu.SemaphoreType.DMA,
            pltpu.SemaphoreType.DMA,
        ],
        compiler_params=pltpu.CompilerParams(use_tc_tiling_on_sc=False),
    )
    return agg, deg


B = 2000


def _ln_relu(z, g, b):
    m = jnp.mean(z, axis=-1, keepdims=True)
    v = jnp.mean((z - m) * (z - m), axis=-1, keepdims=True)
    return jnp.maximum((z - m) / jnp.sqrt(v + 1e-5) * g + b, 0.0)


def _tc_input_body(x_ref, w_ref, b_ref, g_ref, bb_ref, out_ref):
    z = jnp.dot(x_ref[...], w_ref[...],
                preferred_element_type=jnp.float32) + b_ref[...]
    h = _ln_relu(z, g_ref[...], bb_ref[...])
    out_ref[0] = h[:, :HH]
    out_ref[1] = h[:, HH:]


def _tc_layer_body(agg_ref, h_ref, deg_ref, wl_ref, bl_ref, wr_ref,
                   g_ref, bb_ref, out_ref):
    hb = jnp.concatenate([h_ref[0], h_ref[1]], axis=-1)
    ab = jnp.concatenate([agg_ref[0], agg_ref[1]], axis=-1)
    deg = deg_ref[0][:, 0:1] + deg_ref[1][:, 0:1]
    inv = 1.0 / jnp.maximum(deg, 1.0)
    z = (jnp.dot(ab * inv, wl_ref[...], preferred_element_type=jnp.float32)
         + bl_ref[...]
         + jnp.dot(hb, wr_ref[...], preferred_element_type=jnp.float32))
    hn = hb + _ln_relu(z, g_ref[...], bb_ref[...])
    out_ref[0] = hn[:, :HH]
    out_ref[1] = hn[:, HH:]


def _tc_out_body(h_ref, w1_ref, b1_ref, g_ref, bb_ref, w2_ref, b2_ref,
                 w3_ref, b3_ref, out_ref):
    hb = jnp.concatenate([h_ref[0], h_ref[1]], axis=-1)
    z1 = _ln_relu(jnp.dot(hb, w1_ref[...],
                          preferred_element_type=jnp.float32) + b1_ref[...],
                  g_ref[...], bb_ref[...])
    z2 = jnp.maximum(jnp.dot(z1, w2_ref[...],
                             preferred_element_type=jnp.float32)
                     + b2_ref[...], 0.0)
    out_ref[...] = jnp.dot(z2, w3_ref[...],
                           preferred_element_type=jnp.float32) + b3_ref[...]


def _full(shape):
    return pl.BlockSpec(shape, lambda i: tuple(0 for _ in shape))


_GRID = N // B
_h2_spec = pl.BlockSpec((NC, B, HH), lambda i: (0, i, 0))


def _tc_input(x8, w8t, b, g, bb):
    return pl.pallas_call(
        _tc_input_body,
        grid=(_GRID,),
        in_specs=[pl.BlockSpec((B, 8), lambda i: (i, 0)),
                  _full((8, H)), _full((1, H)), _full((1, H)), _full((1, H))],
        out_specs=_h2_spec,
        out_shape=jax.ShapeDtypeStruct((NC, N, HH), jnp.float32),
    )(x8, w8t, b, g, bb)


def _tc_layer(agg2, h2, dega, wlt, bl, wrt, g, bb):
    return pl.pallas_call(
        _tc_layer_body,
        grid=(_GRID,),
        in_specs=[_h2_spec, _h2_spec,
                  pl.BlockSpec((NC, B, 16), lambda i: (0, i, 0)),
                  _full((H, H)), _full((1, H)), _full((H, H)),
                  _full((1, H)), _full((1, H))],
        out_specs=_h2_spec,
        out_shape=jax.ShapeDtypeStruct((NC, N, HH), jnp.float32),
    )(agg2, h2, dega, wlt, bl, wrt, g, bb)


def _tc_output(h2, w1t, b1, g, bb, w2t, b2, w3t, b3):
    return pl.pallas_call(
        _tc_out_body,
        grid=(_GRID,),
        in_specs=[_h2_spec,
                  _full((H, H)), _full((1, H)), _full((1, H)), _full((1, H)),
                  _full((H, HH)), _full((1, HH)),
                  _full((HH, OUT)), _full((1, OUT))],
        out_specs=pl.BlockSpec((B, OUT), lambda i: (i, 0)),
        out_shape=jax.ShapeDtypeStruct((N, OUT), jnp.float32),
    )(h2, w1t, b1, g, bb, w2t, b2, w3t, b3)


def kernel(x, edge_index, params):
    src = edge_index[0]
    dst = edge_index[1]
    pad = E_PAD - E
    src2 = jnp.concatenate([src, jnp.zeros((pad,), jnp.int32)]
                           ).reshape(ROWS128, 128)
    dst2 = jnp.concatenate([dst, jnp.full((pad,), N_DUMMY, jnp.int32)]
                           ).reshape(ROWS128, 128)
    edg2 = jnp.stack([src2, dst2], axis=1).reshape(EDG_ROWS, 128)

    x8 = jnp.pad(x, ((0, 0), (0, 8 - IN)))
    zero32 = jnp.zeros((N_TILE, HH), jnp.float32)
    zero16 = jnp.zeros((N_TILE, 16), jnp.float32)
    ones16 = jnp.ones((128, 16), jnp.float32)

    p = params
    row = lambda a: a.reshape(1, -1)

    sc_agg, sc_deg = _sc_kernels()
    h2 = _tc_input(x8, jnp.pad(p['in_W'], ((0, 0), (0, 8 - IN))).T,
                   row(p['in_b']), row(p['in_ln_g']), row(p['in_ln_b']))
    dega = sc_deg(dst2, ones16, zero16)
    for lp in p['layers']:
        agg2 = sc_agg(h2, edg2, zero32)
        h2 = _tc_layer(agg2, h2, dega, lp['Wl'].T, row(lp['bl']),
                       lp['Wr'].T, row(lp['ln_g']), row(lp['ln_b']))
    return _tc_output(h2, p['out_W1'].T, row(p['out_b1']),
                      row(p['out_ln_g']), row(p['out_ln_b']),
                      p['out_W2'].T, row(p['out_b2']),
                      p['out_W3'].T, row(p['out_b3']))
